# Initial kernel scaffold; baseline (speedup 1.0000x reference)
import jax
import jax.numpy as jnp
from jax.experimental import pallas as pl


def kernel(x, edge_index, W, b):
    raise NotImplementedError("write your pallas kernel here")


# speedup vs baseline: 4.8271x; 4.8271x over previous
import functools

import jax
import jax.numpy as jnp
from jax import lax
from jax.experimental import pallas as pl
from jax.experimental.pallas import tpu as pltpu
from jax.experimental.pallas import tpu_sc as plsc

N_NODES = 10000
N_EDGES = 320000
D = 128

NC = 2
NS = 16
NW = NC * NS

EPW = N_EDGES // NW
CH = 80
NCHUNK = EPW // CH

NPAD = 10240
RPT = NPAD // NS
RCOPIES = RPT // CH


def _sc_body(x_ref, src_ref, dst_ref, agg_out, cnt_out,
             src_v, dst_v, rows_v, ones_v, acc_sh, sem):
    cid = lax.axis_index("c")
    sid = lax.axis_index("s")
    wid = cid * NS + sid
    ebase = wid * EPW

    zeros16 = jnp.zeros((16,), jnp.float32)
    one16 = jnp.full((16,), 1.0, jnp.float32)

    def fill(i, carry):
        for c in range(D // 16):
            rows_v[i, pl.ds(c * 16, 16)] = zeros16
            ones_v[i, pl.ds(c * 16, 16)] = zeros16
        return carry

    lax.fori_loop(0, CH, fill, 0)

    def fill_one(i, carry):
        ones_v[i, pl.ds(0, 16)] = one16
        return carry

    lax.fori_loop(0, CH, fill_one, 0)

    def zero_acc(j, carry):
        pltpu.sync_copy(rows_v, acc_sh.at[pl.ds(sid * RPT + j * CH, CH), :])
        return carry

    lax.fori_loop(0, RCOPIES, zero_acc, 0)
    plsc.subcore_barrier()

    def chunk1(i, carry):
        off = ebase + i * CH
        pltpu.sync_copy(src_ref.at[pl.ds(off, CH)], src_v)
        pltpu.sync_copy(dst_ref.at[pl.ds(off, CH)], dst_v)
        pltpu.async_copy(x_ref.at[src_v], rows_v, sem).wait()
        pltpu.sync_copy(rows_v, acc_sh.at[dst_v], add=True)
        return carry

    lax.fori_loop(0, NCHUNK, chunk1, 0)
    plsc.subcore_barrier()

    def out_copy1(j, carry):
        base = sid * RPT + j * CH
        pltpu.sync_copy(acc_sh.at[pl.ds(base, CH), :],
                        agg_out.at[cid, pl.ds(base, CH), :])
        return carry

    lax.fori_loop(0, RCOPIES, out_copy1, 0)

    def refill(i, carry):
        for c in range(D // 16):
            rows_v[i, pl.ds(c * 16, 16)] = zeros16
        return carry

    lax.fori_loop(0, CH, refill, 0)
    lax.fori_loop(0, RCOPIES, zero_acc, 0)
    plsc.subcore_barrier()

    def chunk2(i, carry):
        off = ebase + i * CH
        pltpu.sync_copy(dst_ref.at[pl.ds(off, CH)], dst_v)
        pltpu.sync_copy(ones_v, acc_sh.at[dst_v], add=True)
        return carry

    lax.fori_loop(0, NCHUNK, chunk2, 0)
    plsc.subcore_barrier()

    def out_copy2(j, carry):
        base = sid * RPT + j * CH
        pltpu.sync_copy(acc_sh.at[pl.ds(base, CH), :],
                        cnt_out.at[cid, pl.ds(base, CH), :])
        return carry

    lax.fori_loop(0, RCOPIES, out_copy2, 0)


@functools.lru_cache(maxsize=1)
def _sc_agg():
    return functools.partial(
        pl.kernel,
        mesh=plsc.VectorSubcoreMesh(core_axis_name="c", subcore_axis_name="s",
                                    num_cores=NC, num_subcores=NS),
        out_type=(
            jax.ShapeDtypeStruct((NC, NPAD, D), jnp.float32),
            jax.ShapeDtypeStruct((NC, NPAD, D), jnp.float32),
        ),
        scratch_types=[
            pltpu.VMEM((CH,), jnp.int32),
            pltpu.VMEM((CH,), jnp.int32),
            pltpu.VMEM((CH, D), jnp.float32),
            pltpu.VMEM((CH, D), jnp.float32),
            pltpu.VMEM_SHARED((NPAD, D), jnp.float32),
            pltpu.SemaphoreType.DMA,
        ],
    )(_sc_body)


BM = 1000


def _tc_body(p_ref, c_ref, x_ref, wt_ref, b_ref, o_ref):
    pa = p_ref[0] + p_ref[1]
    cnt = c_ref[0, :, 0:1] + c_ref[1, :, 0:1]
    inv = jnp.where(cnt > 0, 1.0 / cnt, 0.0)
    h = pa * inv
    out = jnp.dot(h, wt_ref[0:D, :], preferred_element_type=jnp.float32,
                  precision=lax.Precision.HIGHEST)
    out += jnp.dot(x_ref[...], wt_ref[D:2 * D, :],
                   preferred_element_type=jnp.float32,
                   precision=lax.Precision.HIGHEST)
    o_ref[...] = out + b_ref[...]


def _tc_linear(p, c, x, wt, b2):
    return pl.pallas_call(
        _tc_body,
        grid=(N_NODES // BM,),
        in_specs=[
            pl.BlockSpec((NC, BM, D), lambda m: (0, m, 0)),
            pl.BlockSpec((NC, BM, D), lambda m: (0, m, 0)),
            pl.BlockSpec((BM, D), lambda m: (m, 0)),
            pl.BlockSpec((2 * D, D), lambda m: (0, 0)),
            pl.BlockSpec((1, D), lambda m: (0, 0)),
        ],
        out_specs=pl.BlockSpec((BM, D), lambda m: (m, 0)),
        out_shape=jax.ShapeDtypeStruct((N_NODES, D), jnp.float32),
    )(p, c, x, wt, b2)


def kernel(x, edge_index, W, b):
    src = edge_index[0].astype(jnp.int32)
    dst = edge_index[1].astype(jnp.int32)
    p, c = _sc_agg()(x, src, dst)
    wt = W.T
    b2 = b.reshape(1, D)
    return _tc_linear(p, c, x, wt, b2)
